# Initial kernel scaffold; baseline (speedup 1.0000x reference)
import jax
import jax.numpy as jnp
from jax.experimental import pallas as pl


def kernel(x_2d):
    raise NotImplementedError("write your pallas kernel here")


# speedup vs baseline: 3.1241x; 3.1241x over previous
import functools

import jax
import jax.numpy as jnp
import numpy as np
from jax import lax
from jax.experimental import pallas as pl
from jax.experimental.pallas import tpu as pltpu
from jax.experimental.pallas import tpu_sc as plsc

_STRIPE = 4
_DIRECTIONS = ("h_fwd", "h_bwd", "v_fwd", "v_bwd")


def _nss_indices(H, W, stripe_width, direction):
    L = H * W
    indices = np.zeros(L, dtype=np.int64)
    if direction.startswith("h"):
        pos = 0
        num_stripes = (H + stripe_width - 1) // stripe_width
        for s in range(num_stripes):
            row_start = s * stripe_width
            row_end = min(row_start + stripe_width, H)
            for local_r, r in enumerate(range(row_start, row_end)):
                if local_r % 2 == 0:
                    for c in range(W):
                        indices[pos] = r * W + c
                        pos += 1
                else:
                    for c in range(W - 1, -1, -1):
                        indices[pos] = r * W + c
                        pos += 1
        if direction == "h_bwd":
            indices = indices[::-1].copy()
    else:
        pos = 0
        num_stripes = (W + stripe_width - 1) // stripe_width
        for s in range(num_stripes):
            col_start = s * stripe_width
            col_end = min(col_start + stripe_width, W)
            for local_c, c in enumerate(range(col_start, col_end)):
                if local_c % 2 == 0:
                    for r in range(H):
                        indices[pos] = r * W + c
                        pos += 1
                else:
                    for r in range(H - 1, -1, -1):
                        indices[pos] = r * W + c
                        pos += 1
        if direction == "v_bwd":
            indices = indices[::-1].copy()
    return indices


@functools.lru_cache(maxsize=None)
def _global_indices(N, H, W):
    L = H * W
    blocks = []
    for d in _DIRECTIONS:
        idx = _nss_indices(H, W, _STRIPE, d)
        for n in range(N):
            blocks.append(n * L + idx)
    return np.concatenate(blocks).astype(np.int32)


@functools.lru_cache(maxsize=None)
def _make_sc_gather(B, D, b_ch):
    info = plsc.get_sparse_core_info()
    NC, NS = info.num_cores, info.num_subcores
    NW = NC * NS
    per_w = B // NW
    n_ch = per_w // b_ch
    assert per_w % b_ch == 0 and B % NW == 0
    mesh = plsc.VectorSubcoreMesh(core_axis_name="c", subcore_axis_name="s")

    @functools.partial(
        pl.kernel,
        mesh=mesh,
        out_type=jax.ShapeDtypeStruct((B, D), jnp.float32),
        scratch_types=[
            pltpu.VMEM((b_ch,), jnp.int32),
            pltpu.VMEM((b_ch, D), jnp.float32),
            pltpu.SemaphoreType.DMA,
        ],
    )
    def gather_kernel(table_hbm, idx_hbm, out_hbm, idx_v, rows_v, sem):
        wid = lax.axis_index("s") * NC + lax.axis_index("c")
        base0 = wid * per_w

        def body(i, carry):
            base = base0 + i * b_ch
            pltpu.sync_copy(idx_hbm.at[pl.ds(base, b_ch)], idx_v)
            pltpu.async_copy(table_hbm.at[idx_v], rows_v, sem).wait()
            pltpu.sync_copy(rows_v, out_hbm.at[pl.ds(base, b_ch)])
            return carry

        lax.fori_loop(0, n_ch, body, 0)

    return gather_kernel


def kernel(x_2d):
    N, H, W, C = x_2d.shape
    L = H * W
    table = x_2d.reshape(N * L, C)
    gidx = jnp.asarray(_global_indices(N, H, W))
    B = 4 * N * L
    out = _make_sc_gather(B, C, 256)(table, gidx)
    return out.reshape(4 * N, L, C)
